# Initial kernel scaffold; baseline (speedup 1.0000x reference)
import jax
import jax.numpy as jnp
from jax.experimental import pallas as pl


def kernel(atom, bond, adj_matrix, W_C, b_C, W_H, b_H, W_O, b_O, W_N, b_N, W_conv, b_conv, Wb, bb, W_lin, b_lin):
    raise NotImplementedError("write your pallas kernel here")


# speedup vs baseline: 34.4928x; 34.4928x over previous
import functools

import jax
import jax.numpy as jnp
from jax import lax
from jax.experimental import pallas as pl
from jax.experimental.pallas import tpu as pltpu

B, N, M = 1024, 64, 12
ATOM_IN, ATOM_OUT, BOND_IN = 64, 25, 16
G = 4
OUTP = 32
MP = 16
BB = 8


def _body(atom_ref, bond_ref, adj_ref, wstack_ref, bstack_ref, wbblk_ref,
          bbrow_ref, wconv_ref, bconv_ref, wlpad_ref, blinrow_ref, out_ref):
    pieces = []
    for g in range(G):
        a_g = atom_ref[:, g * 16:(g + 1) * 16, :].reshape(BB * 16, ATOM_IN)
        h_g = jnp.dot(a_g, wstack_ref[g], preferred_element_type=jnp.float32)
        h_g = h_g + bstack_ref[g:g + 1, :]
        pieces.append(h_g.reshape(BB, 16, OUTP))
    h = jnp.concatenate(pieces, axis=1)

    bflat = bond_ref[...].reshape(BB * N, M * BOND_IN)
    logits = jnp.dot(bflat, wbblk_ref[...],
                     preferred_element_type=jnp.float32) + bbrow_ref[0:1, :]
    gate = 1.0 / (1.0 + jnp.exp(-logits))
    gate3 = gate.reshape(BB, N, MP)

    iota_k = lax.broadcasted_iota(jnp.int32, (BB, N, N), 2)
    adj = adj_ref[...]
    acc = jnp.zeros((BB, N, N), dtype=jnp.float32)
    for m in range(M):
        hit = adj[:, :, m:m + 1] == iota_k
        acc = acc + jnp.where(hit, gate3[:, :, m:m + 1], 0.0)

    pres = []
    for b in range(BB):
        pres.append(jnp.dot(acc[b], h[b],
                            preferred_element_type=jnp.float32))
    pre = jnp.concatenate([p.reshape(1, N, OUTP) for p in pres], axis=0)

    msg = jnp.dot(pre.reshape(BB * N, OUTP), wconv_ref[...],
                  preferred_element_type=jnp.float32) + bconv_ref[0:1, :]
    au = jnp.maximum(h.reshape(BB * N, OUTP) + msg, 0.0)
    pooled = jnp.sum(au.reshape(BB, N, OUTP), axis=1) * (1.0 / N)
    e = jnp.dot(pooled, wlpad_ref[...],
                preferred_element_type=jnp.float32) + blinrow_ref[0:1, :]
    out_ref[...] = jnp.maximum(e, 0.0)


@functools.partial(jax.jit, static_argnames=())
def kernel(atom, bond, adj_matrix, W_C, b_C, W_H, b_H, W_O, b_O, W_N, b_N,
           W_conv, b_conv, Wb, bb, W_lin, b_lin):
    f32 = jnp.float32
    wstack = jnp.zeros((G, ATOM_IN, OUTP), f32)
    bstack = jnp.zeros((G, OUTP), f32)
    for g, (W, bvec) in enumerate([(W_C, b_C), (W_H, b_H), (W_O, b_O), (W_N, b_N)]):
        wstack = wstack.at[g, :, :ATOM_OUT].set(W)
        bstack = bstack.at[g, :ATOM_OUT].set(bvec)
    wbblk = jnp.zeros((M * BOND_IN, MP), f32)
    for m in range(M):
        wbblk = wbblk.at[m * BOND_IN:(m + 1) * BOND_IN, m].set(Wb[:, 0])
    bbrow = jnp.broadcast_to(bb.astype(f32), (1, MP))
    wconv = jnp.zeros((OUTP, OUTP), f32).at[:ATOM_OUT, :ATOM_OUT].set(W_conv)
    bconv = jnp.zeros((1, OUTP), f32).at[0, :ATOM_OUT].set(b_conv)
    wlpad = jnp.zeros((OUTP, OUTP), f32).at[:ATOM_OUT, 0].set(W_lin[:, 0])
    blinrow = jnp.zeros((1, OUTP), f32).at[0, 0].set(b_lin[0])

    bond2 = bond.reshape(B, N, M * BOND_IN)

    grid = (B // BB,)
    rep = lambda i: (0, 0)
    rep3 = lambda i: (0, 0, 0)
    out2 = pl.pallas_call(
        _body,
        grid=grid,
        in_specs=[
            pl.BlockSpec((BB, N, ATOM_IN), lambda i: (i, 0, 0)),
            pl.BlockSpec((BB, N, M * BOND_IN), lambda i: (i, 0, 0)),
            pl.BlockSpec((BB, N, M), lambda i: (i, 0, 0)),
            pl.BlockSpec((G, ATOM_IN, OUTP), rep3),
            pl.BlockSpec((G, OUTP), rep),
            pl.BlockSpec((M * BOND_IN, MP), rep),
            pl.BlockSpec((1, MP), rep),
            pl.BlockSpec((OUTP, OUTP), rep),
            pl.BlockSpec((1, OUTP), rep),
            pl.BlockSpec((OUTP, OUTP), rep),
            pl.BlockSpec((1, OUTP), rep),
        ],
        out_specs=pl.BlockSpec((BB, OUTP), lambda i: (i, 0)),
        out_shape=jax.ShapeDtypeStruct((B, OUTP), f32),
    )(atom, bond2, adj_matrix, wstack, bstack, wbblk, bbrow, wconv, bconv,
      wlpad, blinrow)
    return out2[:, 0]
